# Initial kernel scaffold; baseline (speedup 1.0000x reference)
import jax
import jax.numpy as jnp
from jax.experimental import pallas as pl


def kernel(x, edge_index, W1, b1, W2, b2, Wc, bc):
    raise NotImplementedError("write your pallas kernel here")


# speedup vs baseline: 18.7681x; 18.7681x over previous
import functools

import jax
import jax.numpy as jnp
from jax import lax
from jax.experimental import pallas as pl
from jax.experimental.pallas import tpu as pltpu
from jax.experimental.pallas import tpu_sc as plsc

_N = 10000
_E = 320000
_NC, _NS = 2, 16
_NW = _NC * _NS
_EPT = _E // _NW
_CH = 80
_NCHUNK = _EPT // _CH
_NPAD = 10240
_RPT = _NPAD // _NS
_RB = 1000
_GRID = _N // _RB


def _mesh():
    return plsc.VectorSubcoreMesh(
        core_axis_name="c", subcore_axis_name="s",
        num_cores=_NC, num_subcores=_NS)


@functools.lru_cache(maxsize=None)
def _build_sc_deg():
    @functools.partial(
        pl.kernel,
        out_type=jax.ShapeDtypeStruct((_NC, _NPAD), jnp.float32),
        mesh=_mesh(),
        compiler_params=pltpu.CompilerParams(use_tc_tiling_on_sc=False),
        scratch_types=[
            pltpu.VMEM((_NCHUNK, _CH), jnp.int32),
            pltpu.VMEM((_CH,), jnp.float32),
            pltpu.VMEM_SHARED((_NPAD,), jnp.float32),
        ],
    )
    def _sc_deg(dst_hbm, zeros_hbm, out_hbm, dst_v, ones_v, acc_sh):
        c = lax.axis_index("c")
        s = lax.axis_index("s")
        w = s * _NC + c
        pltpu.sync_copy(zeros_hbm.at[pl.ds(s * _RPT, _RPT)],
                        acc_sh.at[pl.ds(s * _RPT, _RPT)])
        pltpu.sync_copy(dst_hbm.at[pl.ds(w * _NCHUNK, _NCHUNK)], dst_v)
        for j in range(_CH // 16):
            ones_v[pl.ds(j * 16, 16)] = jnp.ones((16,), jnp.float32)
        plsc.subcore_barrier()

        def body(i, carry):
            pltpu.sync_copy(ones_v, acc_sh.at[dst_v.at[i]], add=True)
            return carry

        lax.fori_loop(0, _NCHUNK, body, 0)
        plsc.subcore_barrier()
        pltpu.sync_copy(acc_sh.at[pl.ds(s * _RPT, _RPT)],
                        out_hbm.at[c, pl.ds(s * _RPT, _RPT)])

    return _sc_deg


@functools.lru_cache(maxsize=None)
def _make_sc_scatter(D):
    @functools.partial(
        pl.kernel,
        out_type=jax.ShapeDtypeStruct((_NC, _NPAD, D), jnp.float32),
        mesh=_mesh(),
        compiler_params=pltpu.CompilerParams(use_tc_tiling_on_sc=False),
        scratch_types=[
            pltpu.VMEM((_NCHUNK, _CH), jnp.int32),
            pltpu.VMEM((_NCHUNK, _CH), jnp.int32),
            pltpu.VMEM((_CH, D), jnp.float32),
            pltpu.VMEM_SHARED((_NPAD, D), jnp.float32),
            pltpu.SemaphoreType.DMA,
        ],
    )
    def sc_scatter(table_hbm, src_hbm, dst_hbm, zeros_hbm, out_hbm,
                   src_v, dst_v, rows_v, acc_sh, sem):
        c = lax.axis_index("c")
        s = lax.axis_index("s")
        w = s * _NC + c
        pltpu.sync_copy(zeros_hbm.at[pl.ds(s * _RPT, _RPT)],
                        acc_sh.at[pl.ds(s * _RPT, _RPT)])
        pltpu.sync_copy(src_hbm.at[pl.ds(w * _NCHUNK, _NCHUNK)], src_v)
        pltpu.sync_copy(dst_hbm.at[pl.ds(w * _NCHUNK, _NCHUNK)], dst_v)
        plsc.subcore_barrier()

        def body(i, carry):
            pltpu.async_copy(table_hbm.at[src_v.at[i]], rows_v, sem).wait()
            pltpu.sync_copy(rows_v, acc_sh.at[dst_v.at[i]], add=True)
            return carry

        lax.fori_loop(0, _NCHUNK, body, 0)
        plsc.subcore_barrier()
        pltpu.sync_copy(acc_sh.at[pl.ds(s * _RPT, _RPT)],
                        out_hbm.at[c, pl.ds(s * _RPT, _RPT)])

    return sc_scatter


def _tck1_body(degT_ref, x_ref, w_ref, hws_ref, dinv_ref):
    deg = degT_ref[:, 0:1] + degT_ref[:, 1:2] + 1.0
    dinv = lax.rsqrt(deg)
    hw = jnp.dot(x_ref[...], w_ref[...], preferred_element_type=jnp.float32)
    hws_ref[...] = hw * dinv
    dinv_ref[...] = dinv


_tck1 = pl.pallas_call(
    _tck1_body,
    grid=(_GRID,),
    in_specs=[
        pl.BlockSpec((_RB, 2), lambda i: (i, 0)),
        pl.BlockSpec((_RB, 128), lambda i: (i, 0)),
        pl.BlockSpec((128, 128), lambda i: (0, 0)),
    ],
    out_specs=[
        pl.BlockSpec((_RB, 128), lambda i: (i, 0)),
        pl.BlockSpec((_RB, 1), lambda i: (i, 0)),
    ],
    out_shape=[
        jax.ShapeDtypeStruct((_N, 128), jnp.float32),
        jax.ShapeDtypeStruct((_N, 1), jnp.float32),
    ],
)


def _make_combine_matmul(dout, relu):
    def body(p_ref, hws_ref, b_ref, dinv_ref, w_ref, h_ref, hwsn_ref):
        agg = p_ref[0] + p_ref[1] + hws_ref[...]
        h = dinv_ref[...] * agg + b_ref[...]
        if relu:
            h = jnp.maximum(h, 0.0)
        h_ref[...] = h
        hwsn_ref[...] = jnp.dot(
            h, w_ref[...], preferred_element_type=jnp.float32) * dinv_ref[...]

    return pl.pallas_call(
        body,
        grid=(_GRID,),
        in_specs=[
            pl.BlockSpec((2, _RB, 128), lambda i: (0, i, 0)),
            pl.BlockSpec((_RB, 128), lambda i: (i, 0)),
            pl.BlockSpec((1, 128), lambda i: (0, 0)),
            pl.BlockSpec((_RB, 1), lambda i: (i, 0)),
            pl.BlockSpec((128, dout), lambda i: (0, 0)),
        ],
        out_specs=[
            pl.BlockSpec((_RB, 128), lambda i: (i, 0)),
            pl.BlockSpec((_RB, dout), lambda i: (i, 0)),
        ],
        out_shape=[
            jax.ShapeDtypeStruct((_N, 128), jnp.float32),
            jax.ShapeDtypeStruct((_N, dout), jnp.float32),
        ],
    )


_tck2 = _make_combine_matmul(128, relu=True)
_tck3 = _make_combine_matmul(16, relu=False)


def _tck4_body(p_ref, hwc_ref, bc_ref, dinv_ref, o_ref):
    agg = p_ref[0] + p_ref[1] + hwc_ref[...]
    o_ref[...] = dinv_ref[...] * agg + bc_ref[...]


_tck4 = pl.pallas_call(
    _tck4_body,
    grid=(_GRID,),
    in_specs=[
        pl.BlockSpec((2, _RB, 16), lambda i: (0, i, 0)),
        pl.BlockSpec((_RB, 16), lambda i: (i, 0)),
        pl.BlockSpec((1, 16), lambda i: (0, 0)),
        pl.BlockSpec((_RB, 1), lambda i: (i, 0)),
    ],
    out_specs=pl.BlockSpec((_RB, 16), lambda i: (i, 0)),
    out_shape=jax.ShapeDtypeStruct((_N, 16), jnp.float32),
)


def kernel(x, edge_index, W1, b1, W2, b2, Wc, bc):
    src2d = edge_index[0].reshape(_E // _CH, _CH)
    dst2d = edge_index[1].reshape(_E // _CH, _CH)
    z128 = jnp.zeros((_NPAD, 128), jnp.float32)
    z16 = jnp.zeros((_NPAD, 16), jnp.float32)
    zdeg = jnp.zeros((_NPAD,), jnp.float32)

    sc_deg = _build_sc_deg()
    sc_scatter128 = _make_sc_scatter(128)
    sc_scatter16 = _make_sc_scatter(16)

    deg_p = sc_deg(dst2d, zdeg)
    degT = deg_p.T

    hws1, dinv = _tck1(degT, x, W1)
    p1 = sc_scatter128(hws1, src2d, dst2d, z128)
    h1, hws2 = _tck2(p1, hws1, b1.reshape(1, 128), dinv, W2)
    p2 = sc_scatter128(hws2, src2d, dst2d, z128)
    h2, hwc = _tck3(p2, hws2, b2.reshape(1, 128), dinv, Wc)
    pc = sc_scatter16(hwc, src2d, dst2d, z16)
    out = _tck4(pc, hwc, bc.reshape(1, 16), dinv)
    return (out, h1, h2)


# speedup vs baseline: 30.5083x; 1.6255x over previous
import functools

import jax
import jax.numpy as jnp
from jax import lax
from jax.experimental import pallas as pl
from jax.experimental.pallas import tpu as pltpu
from jax.experimental.pallas import tpu_sc as plsc

_N = 10000
_E = 320000
_NC, _NS = 2, 16
_NW = _NC * _NS
_EPT = _E // _NW
_CH = 80
_NCHUNK = _EPT // _CH
_SCH = 100
_SNCHUNK = _EPT // _SCH
_NPAD = 10240
_RPT = _NPAD // _NS
_RB = 1000
_GRID = _N // _RB


def _mesh():
    return plsc.VectorSubcoreMesh(
        core_axis_name="c", subcore_axis_name="s",
        num_cores=_NC, num_subcores=_NS)


@functools.lru_cache(maxsize=None)
def _build_sc_deg():
    @functools.partial(
        pl.kernel,
        out_type=jax.ShapeDtypeStruct((_NC, _NPAD), jnp.float32),
        mesh=_mesh(),
        compiler_params=pltpu.CompilerParams(use_tc_tiling_on_sc=False),
        scratch_types=[
            pltpu.VMEM((_NCHUNK, _CH), jnp.int32),
            pltpu.VMEM((_CH,), jnp.float32),
            pltpu.VMEM_SHARED((_NPAD,), jnp.float32),
        ],
    )
    def _sc_deg(dst_hbm, zeros_hbm, out_hbm, dst_v, ones_v, acc_sh):
        c = lax.axis_index("c")
        s = lax.axis_index("s")
        w = s * _NC + c
        pltpu.sync_copy(zeros_hbm.at[pl.ds(s * _RPT, _RPT)],
                        acc_sh.at[pl.ds(s * _RPT, _RPT)])
        pltpu.sync_copy(dst_hbm.at[pl.ds(w * _NCHUNK, _NCHUNK)], dst_v)
        for j in range(_CH // 16):
            ones_v[pl.ds(j * 16, 16)] = jnp.ones((16,), jnp.float32)
        plsc.subcore_barrier()

        def body(i, carry):
            pltpu.sync_copy(ones_v, acc_sh.at[dst_v.at[i]], add=True)
            return carry

        lax.fori_loop(0, _NCHUNK, body, 0)
        plsc.subcore_barrier()
        pltpu.sync_copy(acc_sh.at[pl.ds(s * _RPT, _RPT)],
                        out_hbm.at[c, pl.ds(s * _RPT, _RPT)])

    return _sc_deg


@functools.lru_cache(maxsize=None)
def _make_sc_scatter(D):
    @functools.partial(
        pl.kernel,
        out_type=jax.ShapeDtypeStruct((_NC, _NPAD, D), jnp.float32),
        mesh=_mesh(),
        compiler_params=pltpu.CompilerParams(use_tc_tiling_on_sc=False),
        scratch_types=[
            pltpu.VMEM((_SNCHUNK, _SCH), jnp.int32),
            pltpu.VMEM((_SNCHUNK, _SCH), jnp.int32),
            pltpu.VMEM((_SCH, D), jnp.float32),
            pltpu.VMEM((_SCH, D), jnp.float32),
            pltpu.VMEM_SHARED((_NPAD, D), jnp.float32),
            pltpu.SemaphoreType.DMA,
            pltpu.SemaphoreType.DMA,
        ],
    )
    def sc_scatter(table_hbm, src_hbm, dst_hbm, zeros_hbm, out_hbm,
                   src_v, dst_v, rows0, rows1, acc_sh, sem0, sem1):
        c = lax.axis_index("c")
        s = lax.axis_index("s")
        w = s * _NC + c
        pltpu.sync_copy(zeros_hbm.at[pl.ds(s * _RPT, _RPT)],
                        acc_sh.at[pl.ds(s * _RPT, _RPT)])
        pltpu.sync_copy(src_hbm.at[pl.ds(w * _SNCHUNK, _SNCHUNK)], src_v)
        pltpu.sync_copy(dst_hbm.at[pl.ds(w * _SNCHUNK, _SNCHUNK)], dst_v)
        plsc.subcore_barrier()

        def gstart(i, buf, sem):
            pltpu.async_copy(table_hbm.at[src_v.at[i]], buf, sem)

        def gwait(i, buf, sem):
            pltpu.make_async_copy(table_hbm.at[src_v.at[i]], buf, sem).wait()

        def scat(i, buf):
            pltpu.sync_copy(buf, acc_sh.at[dst_v.at[i]], add=True)

        gstart(0, rows0, sem0)

        def body(j, carry):
            i0 = 2 * j
            gstart(i0 + 1, rows1, sem1)
            gwait(i0, rows0, sem0)
            scat(i0, rows0)
            gstart(i0 + 2, rows0, sem0)
            gwait(i0 + 1, rows1, sem1)
            scat(i0 + 1, rows1)
            return carry

        lax.fori_loop(0, _SNCHUNK // 2 - 1, body, 0)
        i0 = _SNCHUNK - 2
        gstart(i0 + 1, rows1, sem1)
        gwait(i0, rows0, sem0)
        scat(i0, rows0)
        gwait(i0 + 1, rows1, sem1)
        scat(i0 + 1, rows1)
        plsc.subcore_barrier()
        pltpu.sync_copy(acc_sh.at[pl.ds(s * _RPT, _RPT)],
                        out_hbm.at[c, pl.ds(s * _RPT, _RPT)])

    return sc_scatter


def _tck1_body(degT_ref, x_ref, w_ref, hws_ref, dinv_ref):
    deg = degT_ref[:, 0:1] + degT_ref[:, 1:2] + 1.0
    dinv = lax.rsqrt(deg)
    hw = jnp.dot(x_ref[...], w_ref[...], preferred_element_type=jnp.float32)
    hws_ref[...] = hw * dinv
    dinv_ref[...] = dinv


_tck1 = pl.pallas_call(
    _tck1_body,
    grid=(_GRID,),
    in_specs=[
        pl.BlockSpec((_RB, 2), lambda i: (i, 0)),
        pl.BlockSpec((_RB, 128), lambda i: (i, 0)),
        pl.BlockSpec((128, 128), lambda i: (0, 0)),
    ],
    out_specs=[
        pl.BlockSpec((_RB, 128), lambda i: (i, 0)),
        pl.BlockSpec((_RB, 1), lambda i: (i, 0)),
    ],
    out_shape=[
        jax.ShapeDtypeStruct((_N, 128), jnp.float32),
        jax.ShapeDtypeStruct((_N, 1), jnp.float32),
    ],
)


def _make_combine_matmul(dout, relu):
    def body(p_ref, hws_ref, b_ref, dinv_ref, w_ref, h_ref, hwsn_ref):
        agg = p_ref[0] + p_ref[1] + hws_ref[...]
        h = dinv_ref[...] * agg + b_ref[...]
        if relu:
            h = jnp.maximum(h, 0.0)
        h_ref[...] = h
        hwsn_ref[...] = jnp.dot(
            h, w_ref[...], preferred_element_type=jnp.float32) * dinv_ref[...]

    return pl.pallas_call(
        body,
        grid=(_GRID,),
        in_specs=[
            pl.BlockSpec((2, _RB, 128), lambda i: (0, i, 0)),
            pl.BlockSpec((_RB, 128), lambda i: (i, 0)),
            pl.BlockSpec((1, 128), lambda i: (0, 0)),
            pl.BlockSpec((_RB, 1), lambda i: (i, 0)),
            pl.BlockSpec((128, dout), lambda i: (0, 0)),
        ],
        out_specs=[
            pl.BlockSpec((_RB, 128), lambda i: (i, 0)),
            pl.BlockSpec((_RB, dout), lambda i: (i, 0)),
        ],
        out_shape=[
            jax.ShapeDtypeStruct((_N, 128), jnp.float32),
            jax.ShapeDtypeStruct((_N, dout), jnp.float32),
        ],
    )


_tck2 = _make_combine_matmul(128, relu=True)
_tck3 = _make_combine_matmul(16, relu=False)


def _tck4_body(p_ref, hwc_ref, bc_ref, dinv_ref, o_ref):
    agg = p_ref[0] + p_ref[1] + hwc_ref[...]
    o_ref[...] = dinv_ref[...] * agg + bc_ref[...]


_tck4 = pl.pallas_call(
    _tck4_body,
    grid=(_GRID,),
    in_specs=[
        pl.BlockSpec((2, _RB, 16), lambda i: (0, i, 0)),
        pl.BlockSpec((_RB, 16), lambda i: (i, 0)),
        pl.BlockSpec((1, 16), lambda i: (0, 0)),
        pl.BlockSpec((_RB, 1), lambda i: (i, 0)),
    ],
    out_specs=pl.BlockSpec((_RB, 16), lambda i: (i, 0)),
    out_shape=jax.ShapeDtypeStruct((_N, 16), jnp.float32),
)


def kernel(x, edge_index, W1, b1, W2, b2, Wc, bc):
    src2d = edge_index[0].reshape(_E // _SCH, _SCH)
    dst2d = edge_index[1].reshape(_E // _SCH, _SCH)
    dstdeg2d = edge_index[1].reshape(_E // _CH, _CH)
    z128 = jnp.zeros((_NPAD, 128), jnp.float32)
    z16 = jnp.zeros((_NPAD, 16), jnp.float32)
    zdeg = jnp.zeros((_NPAD,), jnp.float32)

    sc_deg = _build_sc_deg()
    sc_scatter128 = _make_sc_scatter(128)
    sc_scatter16 = _make_sc_scatter(16)

    deg_p = sc_deg(dstdeg2d, zdeg)
    degT = deg_p.T

    hws1, dinv = _tck1(degT, x, W1)
    p1 = sc_scatter128(hws1, src2d, dst2d, z128)
    h1, hws2 = _tck2(p1, hws1, b1.reshape(1, 128), dinv, W2)
    p2 = sc_scatter128(hws2, src2d, dst2d, z128)
    h2, hwc = _tck3(p2, hws2, b2.reshape(1, 128), dinv, Wc)
    pc = sc_scatter16(hwc, src2d, dst2d, z16)
    out = _tck4(pc, hwc, bc.reshape(1, 16), dinv)
    return (out, h1, h2)
